# Initial kernel scaffold; baseline (speedup 1.0000x reference)
import jax
import jax.numpy as jnp
from jax.experimental import pallas as pl


def kernel(x, gate_w, w1, w3, w2):
    raise NotImplementedError("write your pallas kernel here")


# speedup vs baseline: 2.6040x; 2.6040x over previous
import functools

import jax
import jax.numpy as jnp
from jax.experimental import pallas as pl
from jax.experimental.pallas import tpu as pltpu

K = 2


def _moe_body(eid_ref, rs_ref, nv_ref, tok_ref, p_ref,
              xf_ref, w1_ref, w3_ref, w2_ref, out_ref,
              xs_ref, acc_ref, *, nh):
    w = pl.program_id(0)
    h = pl.program_id(1)

    @pl.when(jnp.logical_and(w == 0, h == 0))
    def _():
        out_ref[...] = jnp.zeros_like(out_ref)

    nv = nv_ref[w]
    rs = rs_ref[w]

    @pl.when(nv > 0)
    def _():
        @pl.when(h == 0)
        def _():
            def gather_row(i, carry):
                t = tok_ref[rs + i]
                xs_ref[pl.ds(i, 1), :] = xf_ref[pl.ds(t, 1), :]
                return carry
            jax.lax.fori_loop(0, nv, gather_row, 0)

        xb = xs_ref[...].astype(jnp.bfloat16)
        w1b = w1_ref[0].astype(jnp.bfloat16)
        w3b = w3_ref[0].astype(jnp.bfloat16)
        w2b = w2_ref[0].astype(jnp.bfloat16)
        g = jnp.dot(xb, w1b, preferred_element_type=jnp.float32)
        u = jnp.dot(xb, w3b, preferred_element_type=jnp.float32)
        hh = (g * jax.nn.sigmoid(g) * u).astype(jnp.bfloat16)
        part = jnp.dot(hh, w2b, preferred_element_type=jnp.float32)

        @pl.when(h == 0)
        def _():
            acc_ref[...] = part

        @pl.when(h != 0)
        def _():
            acc_ref[...] += part

        @pl.when(h == nh - 1)
        def _():
            def scatter_row(i, carry):
                r = rs + i
                t = tok_ref[r]
                out_ref[pl.ds(t, 1), :] += p_ref[r] * acc_ref[pl.ds(i, 1), :]
                return carry
            jax.lax.fori_loop(0, nv, scatter_row, 0)


def kernel(x, gate_w, w1, w3, w2):
    b, s, d = x.shape
    e_num, _, hid = w1.shape
    t_num = b * s
    r_num = t_num * K
    xf = x.reshape(t_num, d)

    scores = xf @ gate_w.T
    expert_weights, expert_indices = jax.lax.top_k(scores, K)
    expert_weights = jax.nn.softmax(expert_weights, axis=-1)

    ef = expert_indices.reshape(-1).astype(jnp.int32)
    order = jnp.argsort(ef).astype(jnp.int32)
    tok = (order // K).astype(jnp.int32)
    p_sorted = expert_weights.reshape(-1)[order]
    counts = jnp.bincount(ef, length=e_num).astype(jnp.int32)
    starts = (jnp.cumsum(counts) - counts).astype(jnp.int32)

    tm = 512
    maxj = r_num // tm
    w_items = e_num + maxj - 1
    e_c = jnp.repeat(jnp.arange(e_num, dtype=jnp.int32), maxj)
    j_c = jnp.tile(jnp.arange(maxj, dtype=jnp.int32), e_num)
    cnt_c = counts[e_c]
    valid = cnt_c > j_c * tm
    ordc = jnp.argsort(jnp.logical_not(valid).astype(jnp.int32))[:w_items]
    v_w = valid[ordc]
    eidw = jnp.where(v_w, e_c[ordc], e_num - 1).astype(jnp.int32)
    rsw = jnp.where(v_w, starts[e_c[ordc]] + j_c[ordc] * tm, 0).astype(jnp.int32)
    nvw = jnp.clip(cnt_c[ordc] - j_c[ordc] * tm, 0, tm).astype(jnp.int32)

    nh = 4
    hb = hid // nh

    grid_spec = pltpu.PrefetchScalarGridSpec(
        num_scalar_prefetch=5,
        grid=(w_items, nh),
        in_specs=[
            pl.BlockSpec((t_num, d), lambda w, h, eid, rs, nv, tk, p: (0, 0)),
            pl.BlockSpec((1, d, hb), lambda w, h, eid, rs, nv, tk, p: (eid[w], 0, h)),
            pl.BlockSpec((1, d, hb), lambda w, h, eid, rs, nv, tk, p: (eid[w], 0, h)),
            pl.BlockSpec((1, hb, d), lambda w, h, eid, rs, nv, tk, p: (eid[w], h, 0)),
        ],
        out_specs=pl.BlockSpec((t_num, d), lambda w, h, eid, rs, nv, tk, p: (0, 0)),
        scratch_shapes=[
            pltpu.VMEM((tm, d), jnp.float32),
            pltpu.VMEM((tm, d), jnp.float32),
        ],
    )

    out = pl.pallas_call(
        functools.partial(_moe_body, nh=nh),
        grid_spec=grid_spec,
        out_shape=jax.ShapeDtypeStruct((t_num, d), jnp.float32),
        compiler_params=pltpu.CompilerParams(
            dimension_semantics=("arbitrary", "arbitrary"),
            vmem_limit_bytes=100 * 1024 * 1024,
        ),
    )(eidw, rsw, nvw, tok, p_sorted, xf, w1, w3, w2)

    return out.reshape(b, s, d)
